# Initial kernel scaffold; baseline (speedup 1.0000x reference)
import functools
import math

import jax
import jax.numpy as jnp
from jax import lax
from jax.experimental import pallas as pl
from jax.experimental.pallas import tpu as pltpu
from jax.experimental.pallas import tpu_sc as plsc

N_NODES = 10000
N_EDGES = 320000
HIDDEN = 128
N_HEADS = 8
HEAD_DIM = HIDDEN // N_HEADS

NC = 2
NS = 16
NW = NC * NS
PER_W = N_EDGES // NW
C = 80
CHUNKS = PER_W // C
ROWS_PER_TILE = N_NODES // NS


def _proj_body(x_ref, wq_ref, wk_ref, wv_ref, q_ref, k_ref, v_ref):
    xb = x_ref[...]
    q_ref[...] = jnp.dot(xb, wq_ref[...], preferred_element_type=jnp.float32)
    k_ref[...] = jnp.dot(xb, wk_ref[...], preferred_element_type=jnp.float32)
    v_ref[...] = jnp.dot(xb, wv_ref[...], preferred_element_type=jnp.float32)


def _project(x, wq_t, wk_t, wv_t):
    blk = 400
    grid = (N_NODES // blk,)
    out = jax.ShapeDtypeStruct((N_NODES, HIDDEN), jnp.float32)
    w_spec = pl.BlockSpec((HIDDEN, HIDDEN), lambda i: (0, 0))
    return pl.pallas_call(
        _proj_body,
        grid=grid,
        in_specs=[pl.BlockSpec((blk, HIDDEN), lambda i: (i, 0)),
                  w_spec, w_spec, w_spec],
        out_specs=[pl.BlockSpec((blk, HIDDEN), lambda i: (i, 0))] * 3,
        out_shape=[out, out, out],
    )(x, wq_t, wk_t, wv_t)


def _combine_body(p_ref, o_ref):
    o_ref[...] = p_ref[0] + p_ref[1]


def _combine(partials):
    blk = 400
    return pl.pallas_call(
        _combine_body,
        grid=(N_NODES // blk,),
        in_specs=[pl.BlockSpec((NC, blk, HIDDEN), lambda i: (0, i, 0))],
        out_specs=pl.BlockSpec((blk, HIDDEN), lambda i: (i, 0)),
        out_shape=jax.ShapeDtypeStruct((N_NODES, HIDDEN), jnp.float32),
    )(partials)


def _dyn_splat(vec16, lane):
    idx = jnp.full((16,), lane, dtype=jnp.int32)
    return lax.gather(
        vec16, idx[:, None],
        lax.GatherDimensionNumbers(offset_dims=(), collapsed_slice_dims=(0,),
                                   start_index_map=(0,)),
        (1,), mode=lax.GatherScatterMode.PROMISE_IN_BOUNDS)


def _edge_kernel(q, k, v, w_ij, cut, src, dst):
    mesh = plsc.VectorSubcoreMesh(core_axis_name="c", subcore_axis_name="s",
                                  num_cores=NC, num_subcores=NS)

    @functools.partial(
        pl.kernel,
        out_type=jax.ShapeDtypeStruct((NC, N_NODES, HIDDEN), jnp.float32),
        mesh=mesh,
        scratch_types=[
            pltpu.VMEM((C,), jnp.int32),
            pltpu.VMEM((C,), jnp.int32),
            pltpu.VMEM((C,), jnp.float32),
            pltpu.VMEM((C, HIDDEN), jnp.float32),
            pltpu.VMEM((C, HIDDEN), jnp.float32),
            pltpu.VMEM((C, HIDDEN), jnp.float32),
            pltpu.VMEM((C, HIDDEN), jnp.float32),
            pltpu.VMEM((C, HIDDEN), jnp.float32),
            pltpu.VMEM_SHARED((N_NODES, HIDDEN), jnp.float32),
            pltpu.SemaphoreType.DMA,
            pltpu.SemaphoreType.DMA,
            pltpu.SemaphoreType.DMA,
        ],
    )
    def k_fn(q_hbm, k_hbm, v_hbm, w_hbm, cut_hbm, src_hbm, dst_hbm,
             out_hbm,
             isrc, idst, cb, qb, kb, vb, wb, mb, acc,
             sem0, sem1, sem2):
        c = lax.axis_index("c")
        s = lax.axis_index("s")
        wid = c * NS + s
        z16 = jnp.zeros((16,), jnp.float32)

        def zero_mb(i, carry):
            mb[i // 8, pl.ds((i % 8) * 16, 16)] = z16
            return carry
        lax.fori_loop(0, C * 8, zero_mb, None)

        def zero_acc(i, carry):
            pltpu.sync_copy(mb.at[pl.ds(0, 25), :],
                            acc.at[pl.ds(s * ROWS_PER_TILE + i * 25, 25), :])
            return carry
        lax.fori_loop(0, ROWS_PER_TILE // 25, zero_acc, None)
        plsc.subcore_barrier()

        def chunk(t, carry):
            base = wid * PER_W + t * C
            pltpu.sync_copy(src_hbm.at[pl.ds(base, C)], isrc)
            pltpu.sync_copy(dst_hbm.at[pl.ds(base, C)], idst)
            pltpu.sync_copy(cut_hbm.at[pl.ds(base, C)], cb)
            pltpu.sync_copy(w_hbm.at[pl.ds(base, C), :], wb)
            cp0 = pltpu.async_copy(q_hbm.at[idst], qb, sem0)
            cp1 = pltpu.async_copy(k_hbm.at[isrc], kb, sem1)
            cp2 = pltpu.async_copy(v_hbm.at[isrc], vb, sem2)
            cp0.wait()
            cp1.wait()
            cp2.wait()

            def group(g, carry2):
                cvec = cb[pl.ds(g * 16, 16)] * (1.0 / math.sqrt(HEAD_DIM))
                for j in range(16):
                    e = g * 16 + j
                    cut_splat = _dyn_splat(cvec, j)
                    for h in range(N_HEADS):
                        sl = pl.ds(h * HEAD_DIM, HEAD_DIM)
                        tt = qb[e, sl] * wb[e, sl] * kb[e, sl]
                        cs = plsc.cumsum(tt)
                        ssp = _dyn_splat(cs, 15)
                        mb[e, sl] = vb[e, sl] * (ssp * cut_splat)
                return carry2
            lax.fori_loop(0, C // 16, group, None)

            pltpu.sync_copy(mb, acc.at[idst], add=True)
            return carry
        lax.fori_loop(0, CHUNKS, chunk, None)

        plsc.subcore_barrier()
        pltpu.sync_copy(acc.at[pl.ds(s * ROWS_PER_TILE, ROWS_PER_TILE), :],
                        out_hbm.at[c, pl.ds(s * ROWS_PER_TILE, ROWS_PER_TILE), :])

    return k_fn(q, k, v, w_ij, cut, src, dst)


def kernel(x, w_ij, edge_index, cutoff, Wq, Wk, Wv):
    src = edge_index[0].astype(jnp.int32)
    dst = edge_index[1].astype(jnp.int32)
    cut = cutoff.reshape(-1)
    q, k, v = _project(x, Wq.T, Wk.T, Wv.T)
    partials = _edge_kernel(q, k, v, w_ij, cut, src, dst)
    return _combine(partials)


# speedup vs baseline: 44.4935x; 44.4935x over previous
import functools
import math

import jax
import jax.numpy as jnp
from jax import lax
from jax.experimental import pallas as pl
from jax.experimental.pallas import tpu as pltpu
from jax.experimental.pallas import tpu_sc as plsc

N_NODES = 10000
N_EDGES = 320000
HIDDEN = 128
N_HEADS = 8
HEAD_DIM = HIDDEN // N_HEADS

NC = 2
NS = 16
NW = NC * NS
PER_W = N_EDGES // NW
C = 80
CHUNKS = PER_W // C
ROW_BLK = 624
ROW_TAIL = N_NODES - ROW_BLK * NS


def _proj_body(x_ref, wq_ref, wk_ref, wv_ref, q_ref, k_ref, v_ref):
    xb = x_ref[...]
    q_ref[...] = jnp.dot(xb, wq_ref[...], preferred_element_type=jnp.float32)
    k_ref[...] = jnp.dot(xb, wk_ref[...], preferred_element_type=jnp.float32)
    v_ref[...] = jnp.dot(xb, wv_ref[...], preferred_element_type=jnp.float32)


def _project(x, wq_t, wk_t, wv_t):
    blk = 400
    grid = (N_NODES // blk,)
    out = jax.ShapeDtypeStruct((N_NODES, HIDDEN), jnp.float32)
    w_spec = pl.BlockSpec((HIDDEN, HIDDEN), lambda i: (0, 0))
    return pl.pallas_call(
        _proj_body,
        grid=grid,
        in_specs=[pl.BlockSpec((blk, HIDDEN), lambda i: (i, 0)),
                  w_spec, w_spec, w_spec],
        out_specs=[pl.BlockSpec((blk, HIDDEN), lambda i: (i, 0))] * 3,
        out_shape=[out, out, out],
    )(x, wq_t, wk_t, wv_t)


def _combine_body(p_ref, o_ref):
    o_ref[...] = p_ref[0] + p_ref[1]


def _combine(partials):
    blk = 400
    return pl.pallas_call(
        _combine_body,
        grid=(N_NODES // blk,),
        in_specs=[pl.BlockSpec((NC, blk, HIDDEN), lambda i: (0, i, 0))],
        out_specs=pl.BlockSpec((blk, HIDDEN), lambda i: (i, 0)),
        out_shape=jax.ShapeDtypeStruct((N_NODES, HIDDEN), jnp.float32),
    )(partials)


def _dyn_splat(vec16, lane):
    idx = jnp.full((16,), lane, dtype=jnp.int32)
    return lax.gather(
        vec16, idx[:, None],
        lax.GatherDimensionNumbers(offset_dims=(), collapsed_slice_dims=(0,),
                                   start_index_map=(0,)),
        (1,), mode=lax.GatherScatterMode.PROMISE_IN_BOUNDS)


def _edge_kernel(q, k, v, w_ij, cut, src, dst):
    mesh = plsc.VectorSubcoreMesh(core_axis_name="c", subcore_axis_name="s",
                                  num_cores=NC, num_subcores=NS)

    @functools.partial(
        pl.kernel,
        out_type=jax.ShapeDtypeStruct((NC, N_NODES, HIDDEN), jnp.float32),
        mesh=mesh,
        scratch_types=[
            pltpu.VMEM((C,), jnp.int32),
            pltpu.VMEM((C,), jnp.int32),
            pltpu.VMEM((C,), jnp.float32),
            pltpu.VMEM((C, HIDDEN), jnp.float32),
            pltpu.VMEM((C, HIDDEN), jnp.float32),
            pltpu.VMEM((C, HIDDEN), jnp.float32),
            pltpu.VMEM((C, HIDDEN), jnp.float32),
            pltpu.VMEM_SHARED((N_NODES, HIDDEN), jnp.float32),
            pltpu.SemaphoreType.DMA,
            pltpu.SemaphoreType.DMA,
            pltpu.SemaphoreType.DMA,
        ],
        compiler_params=pltpu.CompilerParams(needs_layout_passes=False),
    )
    def k_fn(q_hbm, k_hbm, v_hbm, w_hbm, cut_hbm, src_hbm, dst_hbm,
             out_hbm,
             isrc, idst, cb, qb, kb, vb, wb, acc,
             sem0, sem1, sem2):
        c = lax.axis_index("c")
        s = lax.axis_index("s")
        wid = c * NS + s
        z16 = jnp.zeros((16,), jnp.float32)

        def zero_wb(i, carry):
            wb[i // 8, pl.ds((i % 8) * 16, 16)] = z16
            return carry
        lax.fori_loop(0, C * 8, zero_wb, None)

        def zero_acc(i, carry):
            pltpu.sync_copy(wb.at[pl.ds(0, 48), :],
                            acc.at[pl.ds(s * ROW_BLK + i * 48, 48), :])
            return carry
        lax.fori_loop(0, ROW_BLK // 48, zero_acc, None)

        @pl.when(s == 0)
        def _zero_tail():
            pltpu.sync_copy(wb.at[pl.ds(0, ROW_TAIL), :],
                            acc.at[pl.ds(ROW_BLK * NS, ROW_TAIL), :])
        plsc.subcore_barrier()

        def chunk(t, carry):
            base = wid * PER_W + t * C
            pltpu.sync_copy(src_hbm.at[pl.ds(base, C)], isrc)
            pltpu.sync_copy(dst_hbm.at[pl.ds(base, C)], idst)
            pltpu.sync_copy(cut_hbm.at[pl.ds(base, C)], cb)
            pltpu.sync_copy(w_hbm.at[pl.ds(base, C), :], wb)
            cp0 = pltpu.async_copy(q_hbm.at[idst], qb, sem0)
            cp1 = pltpu.async_copy(k_hbm.at[isrc], kb, sem1)
            cp2 = pltpu.async_copy(v_hbm.at[isrc], vb, sem2)
            cp0.wait()
            cp1.wait()
            cp2.wait()

            def group(g, carry2):
                cvec = cb[pl.ds(g * 16, 16)] * (1.0 / math.sqrt(HEAD_DIM))
                for j in range(16):
                    e = g * 16 + j
                    cut_splat = _dyn_splat(cvec, j)
                    for h in range(N_HEADS):
                        sl = pl.ds(h * HEAD_DIM, HEAD_DIM)
                        tt = qb[e, sl] * wb[e, sl] * kb[e, sl]
                        cs = plsc.cumsum(tt)
                        ssp = _dyn_splat(cs, 15)
                        wb[e, sl] = vb[e, sl] * (ssp * cut_splat)
                return carry2
            lax.fori_loop(0, C // 16, group, None)

            pltpu.sync_copy(wb, acc.at[idst], add=True)
            return carry
        lax.fori_loop(0, CHUNKS, chunk, None)

        plsc.subcore_barrier()
        pltpu.sync_copy(acc.at[pl.ds(s * ROW_BLK, ROW_BLK), :],
                        out_hbm.at[c, pl.ds(s * ROW_BLK, ROW_BLK), :])

        @pl.when(s == 0)
        def _write_tail():
            pltpu.sync_copy(acc.at[pl.ds(ROW_BLK * NS, ROW_TAIL), :],
                            out_hbm.at[c, pl.ds(ROW_BLK * NS, ROW_TAIL), :])

    return k_fn(q, k, v, w_ij, cut, src, dst)


def kernel(x, w_ij, edge_index, cutoff, Wq, Wk, Wv):
    src = edge_index[0].astype(jnp.int32)
    dst = edge_index[1].astype(jnp.int32)
    cut = cutoff.reshape(-1)
    q, k, v = _project(x, Wq.T, Wk.T, Wv.T)
    partials = _edge_kernel(q, k, v, w_ij, cut, src, dst)
    return _combine(partials)
